# baseline (speedup 1.0000x reference)
import functools

import jax
import jax.numpy as jnp
from jax.experimental import pallas as pl
from jax.experimental.pallas import tpu as pltpu

_B = 1024
_V = 100000
_K = 5
_HALF = _B // 2

_RB = 16


def _rowsumexp_body(x_ref, s_ref):
    s_ref[...] = jnp.sum(jnp.exp(x_ref[...]), axis=1, keepdims=True)


def _rowsumexp(x):
    return pl.pallas_call(
        _rowsumexp_body,
        grid=(_B // _RB,),
        in_specs=[pl.BlockSpec((_RB, _V), lambda r: (r, 0))],
        out_specs=pl.BlockSpec((_RB, 1), lambda r: (r, 0)),
        out_shape=jax.ShapeDtypeStruct((_B, 1), jnp.float32),
    )(x)


def _combine_body(g1_ref, g2_ref, gn_ref, anc_ref, s_ref, inst_ref, ans_ref):
    s1 = s_ref[0:1, :]
    s2 = s_ref[1:2, :]
    p1 = jnp.exp(g1_ref[...]) / s1
    p2 = jnp.exp(g2_ref[...]) / s2
    pn = jnp.sum(jnp.exp(gn_ref[...]), axis=0, keepdims=True) / s1
    a = anc_ref[...]
    ans = -jnp.log(p1 + p2 + pn)
    inst = -jnp.log(p1 + p2)
    ans_ref[...] = (jnp.sum(a * ans) / _B).reshape(1, 1)
    inst_ref[...] = (jnp.sum((1.0 - a) * inst) / _B).reshape(1, 1)


def _combine(g1, g2, gn, anc, s):
    return pl.pallas_call(
        _combine_body,
        out_shape=(
            jax.ShapeDtypeStruct((1, 1), jnp.float32),
            jax.ShapeDtypeStruct((1, 1), jnp.float32),
        ),
    )(g1, g2, gn, anc, s)


def kernel(x, y, position, neighbours):
    s = _rowsumexp(x)

    y1 = y[:_HALF]
    y2 = y[_HALF:]
    pos = position[y1]
    anchor = (pos >= 0).astype(jnp.float32)
    pc = jnp.maximum(pos, 0)
    ncols = neighbours[pc]
    rows = jnp.arange(_HALF)
    g1 = x[rows, y1].reshape(1, _HALF)
    g2 = x[rows + _HALF, y2].reshape(1, _HALF)
    gn = x[rows[:, None], ncols].T

    s2d = s.reshape(2, _HALF)
    inst, ans = _combine(g1, g2, gn, anchor.reshape(1, _HALF), s2d)
    return (inst[0, 0], ans[0, 0])


# speedup vs baseline: 1.0309x; 1.0309x over previous
import functools

import jax
import jax.numpy as jnp
from jax.experimental import pallas as pl
from jax.experimental.pallas import tpu as pltpu

_B = 1024
_V = 100000
_K = 5
_HALF = _B // 2

_RB = 8
_NS = 4
_BAND = _B // _NS


def _rowsumexp_body(*refs):
    x_refs = refs[:_NS]
    s_refs = refs[_NS:]
    for xr, sr in zip(x_refs, s_refs):
        sr[...] = jnp.sum(jnp.exp(xr[...]), axis=1, keepdims=True)


def _rowsumexp(x):
    steps = _BAND // _RB
    in_specs = [
        pl.BlockSpec((_RB, _V), lambda r, k=k: (r + k * steps, 0))
        for k in range(_NS)
    ]
    outs = pl.pallas_call(
        _rowsumexp_body,
        grid=(steps,),
        in_specs=in_specs,
        out_specs=[pl.BlockSpec((_RB, 1), lambda r: (r, 0))] * _NS,
        out_shape=[jax.ShapeDtypeStruct((_BAND, 1), jnp.float32)] * _NS,
    )(*([x] * _NS))
    return jnp.concatenate(outs, axis=0)


def _combine_body(g1_ref, g2_ref, gn_ref, anc_ref, s_ref, inst_ref, ans_ref):
    s1 = s_ref[0:1, :]
    s2 = s_ref[1:2, :]
    p1 = jnp.exp(g1_ref[...]) / s1
    p2 = jnp.exp(g2_ref[...]) / s2
    pn = jnp.sum(jnp.exp(gn_ref[...]), axis=0, keepdims=True) / s1
    a = anc_ref[...]
    ans = -jnp.log(p1 + p2 + pn)
    inst = -jnp.log(p1 + p2)
    ans_ref[...] = (jnp.sum(a * ans) / _B).reshape(1, 1)
    inst_ref[...] = (jnp.sum((1.0 - a) * inst) / _B).reshape(1, 1)


def _combine(g1, g2, gn, anc, s):
    return pl.pallas_call(
        _combine_body,
        out_shape=(
            jax.ShapeDtypeStruct((1, 1), jnp.float32),
            jax.ShapeDtypeStruct((1, 1), jnp.float32),
        ),
    )(g1, g2, gn, anc, s)


def kernel(x, y, position, neighbours):
    s = _rowsumexp(x)

    y1 = y[:_HALF]
    y2 = y[_HALF:]
    pos = position[y1]
    anchor = (pos >= 0).astype(jnp.float32)
    pc = jnp.maximum(pos, 0)
    ncols = neighbours[pc]
    rows = jnp.arange(_HALF)
    g1 = x[rows, y1].reshape(1, _HALF)
    g2 = x[rows + _HALF, y2].reshape(1, _HALF)
    gn = x[rows[:, None], ncols].T

    s2d = s.reshape(2, _HALF)
    inst, ans = _combine(g1, g2, gn, anchor.reshape(1, _HALF), s2d)
    return (inst[0, 0], ans[0, 0])
